# Initial kernel scaffold; baseline (speedup 1.0000x reference)
import jax
import jax.numpy as jnp
from jax.experimental import pallas as pl


def kernel(x, table):
    raise NotImplementedError("write your pallas kernel here")


# speedup vs baseline: 1.0145x; 1.0145x over previous
import functools

import jax
import jax.numpy as jnp
from jax import lax
from jax.experimental import pallas as pl
from jax.experimental.pallas import tpu as pltpu
from jax.experimental.pallas import tpu_sc as plsc

NC, NS = 2, 16
NW = NC * NS
CHUNK = 64


def kernel(x, table):
    B, T = x.shape
    V, D = table.shape
    n = B * T
    b_per_w = n // NW
    nchunk = b_per_w // CHUNK
    assert b_per_w * NW == n and nchunk * CHUNK == b_per_w

    idx = x.reshape(NW, nchunk, CHUNK).astype(jnp.int32)

    mesh = plsc.VectorSubcoreMesh(
        core_axis_name="c", subcore_axis_name="s",
        num_cores=NC, num_subcores=NS)

    @functools.partial(
        pl.kernel,
        out_type=jax.ShapeDtypeStruct((n, D), jnp.float32),
        mesh=mesh,
        scratch_types=[
            pltpu.VMEM((nchunk, CHUNK), jnp.int32),
            pltpu.VMEM((CHUNK, D), jnp.float32),
            pltpu.SemaphoreType.DMA,
        ],
        compiler_params=pltpu.CompilerParams(use_tc_tiling_on_sc=False),
    )
    def gather_kernel(idx_hbm, table_hbm, out_hbm, idx_v, rows_v, sem):
        wid = lax.axis_index("s") * NC + lax.axis_index("c")
        pltpu.sync_copy(idx_hbm.at[wid], idx_v)
        base = wid * b_per_w

        @pl.loop(0, nchunk)
        def _chunk(j):
            pltpu.async_copy(table_hbm.at[idx_v.at[j]], rows_v, sem).wait()
            pltpu.sync_copy(rows_v, out_hbm.at[pl.ds(base + j * CHUNK, CHUNK)])

    out = gather_kernel(idx, table)
    return out.reshape(B, T, D)


# speedup vs baseline: 1.0360x; 1.0212x over previous
import functools

import jax
import jax.numpy as jnp
from jax import lax
from jax.experimental import pallas as pl
from jax.experimental.pallas import tpu as pltpu
from jax.experimental.pallas import tpu_sc as plsc

NC, NS = 2, 16
NW = NC * NS
CHUNK = 40
NBUF = 2


def kernel(x, table):
    B, T = x.shape
    V, D = table.shape
    n = B * T
    b_per_w = n // NW
    nchunk = b_per_w // CHUNK
    assert b_per_w * NW == n and nchunk * CHUNK == b_per_w

    idx = x.reshape(NW, nchunk, CHUNK).astype(jnp.int32)

    mesh = plsc.VectorSubcoreMesh(
        core_axis_name="c", subcore_axis_name="s",
        num_cores=NC, num_subcores=NS)

    @functools.partial(
        pl.kernel,
        out_type=jax.ShapeDtypeStruct((n, D), jnp.float32),
        mesh=mesh,
        scratch_types=[
            pltpu.VMEM((nchunk, CHUNK), jnp.int32),
            [pltpu.VMEM((CHUNK, D), jnp.float32) for _ in range(NBUF)],
            [pltpu.SemaphoreType.DMA for _ in range(NBUF)],
        ],
        compiler_params=pltpu.CompilerParams(use_tc_tiling_on_sc=False),
    )
    def gather_kernel(idx_hbm, table_hbm, out_hbm, idx_v, rows, sems):
        wid = lax.axis_index("s") * NC + lax.axis_index("c")
        pltpu.sync_copy(idx_hbm.at[wid], idx_v)
        base = wid * b_per_w

        for b in range(NBUF):
            pltpu.async_copy(table_hbm.at[idx_v.at[b]], rows[b], sems[b])

        @pl.loop(0, nchunk, step=NBUF)
        def _chunk(j0):
            for b in range(NBUF):
                j = j0 + b
                pltpu.make_async_copy(table_hbm.at[idx_v.at[j]],
                                      rows[b], sems[b]).wait()
                pltpu.sync_copy(rows[b],
                                out_hbm.at[pl.ds(base + j * CHUNK, CHUNK)])

                @pl.when(j + NBUF < nchunk)
                def _refill():
                    pltpu.async_copy(table_hbm.at[idx_v.at[j + NBUF]],
                                     rows[b], sems[b])

    out = gather_kernel(idx, table)
    return out.reshape(B, T, D)


# speedup vs baseline: 1.0371x; 1.0011x over previous
import functools

import jax
import jax.numpy as jnp
from jax import lax
from jax.experimental import pallas as pl
from jax.experimental.pallas import tpu as pltpu
from jax.experimental.pallas import tpu_sc as plsc

NC, NS = 2, 16
NW = NC * NS
NBUF = 2


def kernel(x, table):
    B, T = x.shape
    V, D = table.shape
    nb = B // NW
    assert nb * NW == B and nb % NBUF == 0

    idx = x.reshape(NW, nb, T).astype(jnp.int32)

    mesh = plsc.VectorSubcoreMesh(
        core_axis_name="c", subcore_axis_name="s",
        num_cores=NC, num_subcores=NS)

    @functools.partial(
        pl.kernel,
        out_type=jax.ShapeDtypeStruct((B, T, D), jnp.float32),
        mesh=mesh,
        scratch_types=[
            pltpu.VMEM((nb, T), jnp.int32),
            [pltpu.VMEM((T, D), jnp.float32) for _ in range(NBUF)],
            [pltpu.SemaphoreType.DMA for _ in range(NBUF)],
        ],
        compiler_params=pltpu.CompilerParams(use_tc_tiling_on_sc=False),
    )
    def gather_kernel(idx_hbm, table_hbm, out_hbm, idx_v, rows, sems):
        wid = lax.axis_index("s") * NC + lax.axis_index("c")
        pltpu.sync_copy(idx_hbm.at[wid], idx_v)
        base = wid * nb

        for b in range(NBUF):
            pltpu.async_copy(table_hbm.at[idx_v.at[b]], rows[b], sems[b])

        @pl.loop(0, nb, step=NBUF)
        def _row(j0):
            for b in range(NBUF):
                j = j0 + b
                pltpu.make_async_copy(table_hbm.at[idx_v.at[j]],
                                      rows[b], sems[b]).wait()
                pltpu.sync_copy(rows[b], out_hbm.at[base + j])

                @pl.when(j + NBUF < nb)
                def _refill():
                    pltpu.async_copy(table_hbm.at[idx_v.at[j + NBUF]],
                                     rows[b], sems[b])

    return gather_kernel(idx, table)


# speedup vs baseline: 3.0556x; 2.9463x over previous
import functools

import jax
import jax.numpy as jnp
from jax import lax
from jax.experimental import pallas as pl
from jax.experimental.pallas import tpu as pltpu
from jax.experimental.pallas import tpu_sc as plsc

NC, NS = 2, 16
NW = NC * NS
L = 16


def kernel(x, table):
    B, T = x.shape
    V, D = table.shape
    NVT = D // 8
    NBT = B // 128
    units = -(-NVT // NW)

    xT = x.T.astype(jnp.int32)
    tT = table.T

    mesh = plsc.VectorSubcoreMesh(
        core_axis_name="c", subcore_axis_name="s",
        num_cores=NC, num_subcores=NS)

    @functools.partial(
        pl.kernel,
        out_type=jax.ShapeDtypeStruct((T, NVT, NBT, 8, 128), jnp.float32),
        mesh=mesh,
        scratch_types=[
            pltpu.VMEM((T, B), jnp.int32),
            pltpu.VMEM((8, V), jnp.float32),
            [pltpu.VMEM((NBT, 8, 128), jnp.float32) for _ in range(2)],
            [pltpu.SemaphoreType.DMA for _ in range(2)],
        ],
        compiler_params=pltpu.CompilerParams(
            use_tc_tiling_on_sc=False, needs_layout_passes=False),
    )
    def gather_kernel(xT_hbm, tT_hbm, out_hbm, xv, r8, ov, sems):
        wid = lax.axis_index("s") * NC + lax.axis_index("c")
        pltpu.sync_copy(xT_hbm, xv)

        vs_idx = [jnp.full((L,), vs, jnp.int32) for vs in range(8)]

        @pl.loop(0, units)
        def _unit(u):
            vt = u * NW + wid

            @pl.when(vt < NVT)
            def _do_unit():
                pltpu.sync_copy(tT_hbm.at[pl.ds(vt * 8, 8)], r8)

                @pl.loop(0, T, step=2)
                def _t(t0):
                    for p in range(2):
                        t = t0 + p
                        buf, sem = ov[p], sems[p]
                        @pl.when(t >= 2)
                        def _drain():
                            pltpu.make_async_copy(
                                buf, out_hbm.at[t - 2, vt], sem).wait()

                        @plsc.parallel_loop(0, NBT * 8, unroll=8)
                        def _bc(bc):
                            bt = bc >> 3
                            bb = bc & 7
                            idx16 = xv[t, pl.ds(bc * L, L)]
                            vals = [plsc.load_gather(r8, [vs_idx[vs], idx16])
                                    for vs in range(8)]
                            for vs in range(8):
                                buf[bt, vs, pl.ds(bb * L, L)] = vals[vs]

                        pltpu.async_copy(buf, out_hbm.at[t, vt], sem)

                for p in range(2):
                    pltpu.make_async_copy(
                        ov[p], out_hbm.at[T - 2 + p, vt], sems[p]).wait()

    k5 = gather_kernel(xT, tT)
    return jnp.transpose(k5, (2, 4, 0, 1, 3)).reshape(B, T, D)


# speedup vs baseline: 5.3611x; 1.7545x over previous
import functools

import jax
import jax.numpy as jnp
from jax import lax
from jax.experimental import pallas as pl
from jax.experimental.pallas import tpu as pltpu
from jax.experimental.pallas import tpu_sc as plsc

NC, NS = 2, 16
NW = NC * NS
L = 16


def kernel(x, table):
    B, T = x.shape
    V, D = table.shape
    NVT = D // 8
    NBT = B // 128
    units = -(-NVT // NW)

    xT = x.T.astype(jnp.int32)
    tT = table.T

    mesh = plsc.VectorSubcoreMesh(
        core_axis_name="c", subcore_axis_name="s",
        num_cores=NC, num_subcores=NS)

    @functools.partial(
        pl.kernel,
        out_type=jax.ShapeDtypeStruct((T, NVT, NBT, 8, 128), jnp.float32),
        mesh=mesh,
        scratch_types=[
            pltpu.VMEM((T, B), jnp.int32),
            pltpu.VMEM((8, V), jnp.float32),
            [pltpu.VMEM((NBT, 8, 128), jnp.float32) for _ in range(2)],
            [pltpu.SemaphoreType.DMA for _ in range(2)],
        ],
        compiler_params=pltpu.CompilerParams(
            use_tc_tiling_on_sc=False, needs_layout_passes=False),
    )
    def gather_kernel(xT_hbm, tT_hbm, out_hbm, xv, r8, ov, sems):
        wid = lax.axis_index("s") * NC + lax.axis_index("c")
        pltpu.sync_copy(xT_hbm, xv)

        vs_idx = [jnp.full((L,), vs, jnp.int32) for vs in range(8)]

        @pl.loop(0, units)
        def _unit(u):
            vt = u * NW + wid

            @pl.when(vt < NVT)
            def _do_unit():
                pltpu.sync_copy(tT_hbm.at[pl.ds(vt * 8, 8)], r8)

                @pl.loop(0, T, step=2)
                def _t(t0):
                    for p in range(2):
                        t = t0 + p
                        buf, sem = ov[p], sems[p]
                        @pl.when(t >= 2)
                        def _drain():
                            pltpu.make_async_copy(
                                buf, out_hbm.at[t - 2, vt], sem).wait()

                        @plsc.parallel_loop(0, NBT * 8, unroll=4)
                        def _bc(bc):
                            bt = bc >> 3
                            bb = bc & 7
                            idx16 = xv[t, pl.ds(bc * L, L)]
                            vals = [plsc.load_gather(r8, [vs_idx[vs], idx16])
                                    for vs in range(8)]
                            for vs in range(8):
                                buf[bt, vs, pl.ds(bb * L, L)] = vals[vs]

                        pltpu.async_copy(buf, out_hbm.at[t, vt], sem)

                for p in range(2):
                    pltpu.make_async_copy(
                        ov[p], out_hbm.at[T - 2 + p, vt], sems[p]).wait()

    k5 = gather_kernel(xT, tT)
    return jnp.transpose(k5, (2, 4, 0, 1, 3)).reshape(B, T, D)


# speedup vs baseline: 5.7006x; 1.0633x over previous
import functools

import jax
import jax.numpy as jnp
from jax import lax
from jax.experimental import pallas as pl
from jax.experimental.pallas import tpu as pltpu
from jax.experimental.pallas import tpu_sc as plsc

NC, NS = 2, 16
NW = NC * NS
L = 16


def kernel(x, table):
    B, T = x.shape
    V, D = table.shape
    NVT = D // 8
    NBT = B // 128
    units = -(-NVT // NW)

    xT = x.T.astype(jnp.int32)
    tT = table.T

    mesh = plsc.VectorSubcoreMesh(
        core_axis_name="c", subcore_axis_name="s",
        num_cores=NC, num_subcores=NS)

    @functools.partial(
        pl.kernel,
        out_type=jax.ShapeDtypeStruct((T, NVT, NBT, 8, 128), jnp.float32),
        mesh=mesh,
        scratch_types=[
            pltpu.VMEM((T, B), jnp.int32),
            pltpu.VMEM((8, V), jnp.float32),
            [pltpu.VMEM((NBT, 8, 128), jnp.float32) for _ in range(2)],
            [pltpu.SemaphoreType.DMA for _ in range(2)],
        ],
        compiler_params=pltpu.CompilerParams(
            use_tc_tiling_on_sc=False, needs_layout_passes=False),
    )
    def gather_kernel(xT_hbm, tT_hbm, out_hbm, xv, r8, ov, sems):
        wid = lax.axis_index("s") * NC + lax.axis_index("c")
        pltpu.sync_copy(xT_hbm, xv)

        vs_idx = [jnp.full((L,), vs, jnp.int32) for vs in range(8)]

        @pl.loop(0, units)
        def _unit(u):
            vt = u * NW + wid

            @pl.when(vt < NVT)
            def _do_unit():
                pltpu.sync_copy(tT_hbm.at[pl.ds(vt * 8, 8)], r8)

                @pl.loop(0, T, step=2)
                def _t(t0):
                    for p in range(2):
                        t = t0 + p
                        buf, sem = ov[p], sems[p]
                        @pl.when(t >= 2)
                        def _drain():
                            pltpu.make_async_copy(
                                buf, out_hbm.at[t - 2, vt], sem).wait()

                        @plsc.parallel_loop(0, NBT * 8, unroll=2)
                        def _bc(bc):
                            bt = bc >> 3
                            bb = bc & 7
                            idx16 = xv[t, pl.ds(bc * L, L)]
                            vals = [plsc.load_gather(r8, [vs_idx[vs], idx16])
                                    for vs in range(8)]
                            for vs in range(8):
                                buf[bt, vs, pl.ds(bb * L, L)] = vals[vs]

                        pltpu.async_copy(buf, out_hbm.at[t, vt], sem)

                for p in range(2):
                    pltpu.make_async_copy(
                        ov[p], out_hbm.at[T - 2 + p, vt], sems[p]).wait()

    k5 = gather_kernel(xT, tT)
    return jnp.transpose(k5, (2, 4, 0, 1, 3)).reshape(B, T, D)
